# Initial kernel scaffold; baseline (speedup 1.0000x reference)
import jax
import jax.numpy as jnp
from jax.experimental import pallas as pl


def kernel(output, anchors, targets):
    raise NotImplementedError("write your pallas kernel here")


# speedup vs baseline: 6.0037x; 6.0037x over previous
import functools

import jax
import jax.numpy as jnp
from jax import lax
from jax.experimental import pallas as pl
from jax.experimental.pallas import tpu as pltpu
from jax.experimental.pallas import tpu_sc as plsc

_B, _A, _H, _W = 8, 9, 128, 128
_T = 128
_L = 16
_NCH = _T // _L
_DUMP = _A * _H * _W


def _rsqrt(x):
    i = plsc.bitcast(x, jnp.int32)
    i = 0x5F3759DF - jnp.right_shift(i, 1)
    y = plsc.bitcast(i, jnp.float32)
    for _ in range(3):
        y = y * (1.5 - 0.5 * x * y * y)
    return y


def _main_image(active, b, out_hbm, anch_hbm, tgt_hbm,
                tgt_ref, anch_ref, cell_ref, sbuf_ref, mask_ref,
                pred_ref, loss_ref, part_shr, sem):
    @pl.when(active)
    def _main():
        pltpu.sync_copy(tgt_hbm.at[b], tgt_ref)
        pltpu.sync_copy(anch_hbm, anch_ref)

        lanes = lax.iota(jnp.int32, 16)
        arow = jnp.clip(lanes, 0, _A - 1) * 2
        aw_all = plsc.load_gather(anch_ref, [arow])
        ah_all = plsc.load_gather(anch_ref, [arow + 1])
        ax1s, ay1s, areas = [], [], []
        for a in range(_A):
            awv = jnp.broadcast_to(aw_all[a], (16,))
            ahv = jnp.broadcast_to(ah_all[a], (16,))
            ax1 = awv * 0.5
            ay1 = ahv * 0.5
            ax0 = 0.0 - ax1
            ay0 = 0.0 - ay1
            ax1s.append(ax1)
            ay1s.append(ay1)
            areas.append((ax1 - ax0) * (ay1 - ay0))
        for c in range(_NCH):
            gidx = lanes + (c * _L)
            cols = [jnp.full((16,), k, jnp.int32) for k in range(1, 5)]
            t1 = plsc.load_gather(tgt_ref, [gidx, cols[0]])
            t2 = plsc.load_gather(tgt_ref, [gidx, cols[1]])
            t3 = plsc.load_gather(tgt_ref, [gidx, cols[2]])
            t4 = plsc.load_gather(tgt_ref, [gidx, cols[3]])
            valid = jnp.logical_not(
                (t1 == 0.0) & (t2 == 0.0) & (t3 == 0.0) & (t4 == 0.0))
            sx = t1 * float(_W)
            sy = t2 * float(_H)
            sw = t3 * float(_W)
            sh = t4 * float(_H)
            cxi = sx.astype(jnp.int32)
            cyi = sy.astype(jnp.int32)
            cxf = cxi.astype(jnp.float32)
            cyf = cyi.astype(jnp.float32)
            ox = sx - (cxf + 0.5)
            oy = sy - (cyf + 0.5)
            hw = sw * 0.5
            hh = sh * 0.5
            tx0 = ox - hw
            ty0 = oy - hh
            tx1 = ox + hw
            ty1 = oy + hh
            area_t = (tx1 - tx0) * (ty1 - ty0)
            best = jnp.full((16,), -jnp.inf, jnp.float32)
            arg = jnp.zeros((16,), jnp.int32)
            for a in range(_A):
                ax1 = ax1s[a]
                ay1 = ay1s[a]
                ax0 = 0.0 - ax1
                ay0 = 0.0 - ay1
                area_a = areas[a]
                x0 = jnp.maximum(tx0, ax0)
                y0 = jnp.maximum(ty0, ay0)
                x1 = jnp.minimum(tx1, ax1)
                y1 = jnp.minimum(ty1, ay1)
                flag = ((x0 < x1) & (y0 < y1)).astype(jnp.float32)
                inter = (x1 - x0) * (y1 - y0) * flag
                comb = area_t + area_a
                iou = inter / (comb - inter)
                upd = iou > best
                arg = jnp.where(upd, a, arg)
                best = jnp.where(upd, iou, best)
            maskv = (best > -1.0) & valid
            cell = (arg * _H + cyi) * _W + cxi
            cell = jnp.where(maskv, cell, _DUMP)
            sl = pl.ds(c * _L, _L)
            cell_ref[sl] = cell
            mask_ref[sl] = maskv.astype(jnp.int32)
            sbuf_ref[0, sl] = sx
            sbuf_ref[1, sl] = sy
            sbuf_ref[2, sl] = sw
            sbuf_ref[3, sl] = sh
            argc = jnp.clip(arg, 0, _A - 1)
            cyc = jnp.clip(cyi, 0, _H - 1)
            cxc = jnp.clip(cxi, 0, _W - 1)
            for j in range(_L):
                pltpu.async_copy(
                    out_hbm.at[b, argc[j], cyc[j], cxc[j]],
                    pred_ref.at[c * _L + j], sem)

        cells = [cell_ref[pl.ds(ci * _L, _L)] for ci in range(_NCH)]
        killed = [jnp.zeros((16,), jnp.bool_) for _ in range(_NCH)]
        for cj in range(_NCH):
            cv = cells[cj]
            for q in range(_L):
                bq = jnp.broadcast_to(cv[q], (16,))
                killed[cj] = killed[cj] | ((cv == bq) & (lanes < q))
                for ci in range(cj):
                    killed[ci] = killed[ci] | (cells[ci] == bq)
        for ci in range(_NCH):
            sl = pl.ds(ci * _L, _L)
            win_c = (mask_ref[sl] != 0) & jnp.logical_not(killed[ci])
            mask_ref[sl] = win_c.astype(jnp.int32)

        def _drain(i, carry):
            pltpu.make_async_copy(out_hbm.at[0, 0, 0, 0], pred_ref.at[i],
                                  sem).wait()
            return carry
        lax.fori_loop(0, _T, _drain, 0)

        acc = jnp.zeros((16,), jnp.float32)
        cnt = jnp.zeros((16,), jnp.int32)
        for c in range(_NCH):
            sl = pl.ds(c * _L, _L)
            gidx = lanes + (c * _L)
            win = mask_ref[sl] != 0
            sx = sbuf_ref[0, sl]
            sy = sbuf_ref[1, sl]
            sw = sbuf_ref[2, sl]
            sh = sbuf_ref[3, sl]
            p0 = plsc.load_gather(pred_ref, [gidx, jnp.full((16,), 0, jnp.int32)])
            p1 = plsc.load_gather(pred_ref, [gidx, jnp.full((16,), 1, jnp.int32)])
            p2 = plsc.load_gather(pred_ref, [gidx, jnp.full((16,), 2, jnp.int32)])
            p3 = plsc.load_gather(pred_ref, [gidx, jnp.full((16,), 3, jnp.int32)])
            d0 = p0 - sx
            d1 = p1 - sy
            d2 = _rsqrt(p2) - _rsqrt(sw)
            d3 = _rsqrt(p3) - _rsqrt(sh)
            contrib = d0 * d0 + d1 * d1 + d2 * d2 + d3 * d3
            acc = acc + jnp.where(win, contrib, 0.0)
            cnt = cnt + win.astype(jnp.int32)

        tot_v = jnp.broadcast_to(jnp.sum(acc), (16,))
        n_v = jnp.broadcast_to(jnp.sum(cnt), (16,)).astype(jnp.float32)
        n_v = jnp.maximum(n_v, 1.0)
        loss_ref[...] = tot_v / (2.0 * n_v)
        pltpu.sync_copy(loss_ref, part_shr.at[pl.ds(b * 16, 16)])


def _body(out_hbm, anch_hbm, tgt_hbm, res_hbm,
          tgt_ref, anch_ref, cell_ref, sbuf_ref, mask_ref,
          pred_ref, loss_ref, part_shr, tmp_ref, res_ref, sem):
    cid = lax.axis_index("c")
    sid = lax.axis_index("s")
    active = jnp.logical_and(cid == 0, sid < _B)

    _main_image(active, sid, out_hbm, anch_hbm, tgt_hbm,
                tgt_ref, anch_ref, cell_ref, sbuf_ref, mask_ref,
                pred_ref, loss_ref, part_shr, sem)

    plsc.subcore_barrier()

    @pl.when(jnp.logical_and(cid == 0, sid == 0))
    def _reduce():
        facc = jnp.zeros((16,), jnp.float32)
        for bb in range(_B):
            pltpu.sync_copy(part_shr.at[pl.ds(bb * 16, 16)], tmp_ref)
            facc = facc + tmp_ref[...]
        res_ref[...] = facc * (1.0 / _B)
        pltpu.sync_copy(res_ref, res_hbm)


def kernel(output, anchors, targets):
    anch_flat = jnp.pad(anchors.reshape(-1), (0, 24 - 2 * _A))
    mesh = plsc.VectorSubcoreMesh(core_axis_name="c", subcore_axis_name="s")
    fn = functools.partial(
        pl.kernel,
        out_type=jax.ShapeDtypeStruct((16,), jnp.float32),
        mesh=mesh,
        compiler_params=pltpu.CompilerParams(needs_layout_passes=False),
        scratch_types=[
            pltpu.VMEM((_T, 5), jnp.float32),
            pltpu.VMEM((24,), jnp.float32),
            pltpu.VMEM((_T,), jnp.int32),
            pltpu.VMEM((4, _T), jnp.float32),
            pltpu.VMEM((_T,), jnp.int32),
            pltpu.VMEM((_T, 5), jnp.float32),
            pltpu.VMEM((16,), jnp.float32),
            pltpu.VMEM_SHARED((_B * 16,), jnp.float32),
            pltpu.VMEM((16,), jnp.float32),
            pltpu.VMEM((16,), jnp.float32),
            pltpu.SemaphoreType.DMA,
        ],
    )(_body)
    res = fn(output, anch_flat, targets)
    return res[0]


# speedup vs baseline: 69.8418x; 11.6331x over previous
import functools

import jax
import jax.numpy as jnp
from jax import lax
from jax.experimental import pallas as pl
from jax.experimental.pallas import tpu as pltpu
from jax.experimental.pallas import tpu_sc as plsc

_B, _A, _H, _W = 8, 9, 128, 128
_T = 128
_L = 16
_NCH = _T // _L
_DUMP = _A * _H * _W


def _rsqrt(x):
    i = plsc.bitcast(x, jnp.int32)
    i = 0x5F3759DF - jnp.right_shift(i, 1)
    y = plsc.bitcast(i, jnp.float32)
    for _ in range(3):
        y = y * (1.5 - 0.5 * x * y * y)
    return y


def _main_image(active, b, out_hbm, anch_hbm, tgt_hbm,
                tgt_ref, anch_ref, cell_ref, sbuf_ref, mask_ref,
                idx_ref, xbuf_ref, rowbuf_ref, loss_ref, part_shr, sem):
    @pl.when(active)
    def _main():
        pltpu.sync_copy(tgt_hbm.at[pl.ds(0, 5), b], tgt_ref)
        pltpu.sync_copy(anch_hbm, anch_ref)

        lanes = lax.iota(jnp.int32, 16)
        aw_all = anch_ref[0, pl.ds(0, 16)]
        ah_all = anch_ref[1, pl.ds(0, 16)]
        ax1s, ay1s, areas = [], [], []
        for a in range(_A):
            awv = jnp.broadcast_to(aw_all[a], (16,))
            ahv = jnp.broadcast_to(ah_all[a], (16,))
            ax1 = awv * 0.5
            ay1 = ahv * 0.5
            ax0 = 0.0 - ax1
            ay0 = 0.0 - ay1
            ax1s.append(ax1)
            ay1s.append(ay1)
            areas.append((ax1 - ax0) * (ay1 - ay0))
        for c in range(_NCH):
            sl = pl.ds(c * _L, _L)
            t1 = tgt_ref[1, sl]
            t2 = tgt_ref[2, sl]
            t3 = tgt_ref[3, sl]
            t4 = tgt_ref[4, sl]
            valid = jnp.logical_not(
                (t1 == 0.0) & (t2 == 0.0) & (t3 == 0.0) & (t4 == 0.0))
            sx = t1 * float(_W)
            sy = t2 * float(_H)
            sw = t3 * float(_W)
            sh = t4 * float(_H)
            cxi = sx.astype(jnp.int32)
            cyi = sy.astype(jnp.int32)
            cxf = cxi.astype(jnp.float32)
            cyf = cyi.astype(jnp.float32)
            ox = sx - (cxf + 0.5)
            oy = sy - (cyf + 0.5)
            hw = sw * 0.5
            hh = sh * 0.5
            tx0 = ox - hw
            ty0 = oy - hh
            tx1 = ox + hw
            ty1 = oy + hh
            area_t = (tx1 - tx0) * (ty1 - ty0)
            best = jnp.full((16,), -jnp.inf, jnp.float32)
            arg = jnp.zeros((16,), jnp.int32)
            for a in range(_A):
                ax1 = ax1s[a]
                ay1 = ay1s[a]
                ax0 = 0.0 - ax1
                ay0 = 0.0 - ay1
                area_a = areas[a]
                x0 = jnp.maximum(tx0, ax0)
                y0 = jnp.maximum(ty0, ay0)
                x1 = jnp.minimum(tx1, ax1)
                y1 = jnp.minimum(ty1, ay1)
                flag = ((x0 < x1) & (y0 < y1)).astype(jnp.float32)
                inter = (x1 - x0) * (y1 - y0) * flag
                comb = area_t + area_a
                iou = inter / (comb - inter)
                upd = iou > best
                arg = jnp.where(upd, a, arg)
                best = jnp.where(upd, iou, best)
            maskv = (best > -1.0) & valid
            cell = (arg * _H + cyi) * _W + cxi
            cell = jnp.where(maskv, cell, _DUMP)
            cell_ref[sl] = cell
            mask_ref[sl] = maskv.astype(jnp.int32)
            sbuf_ref[0, sl] = sx
            sbuf_ref[1, sl] = sy
            sbuf_ref[2, sl] = sw
            sbuf_ref[3, sl] = sh
            argc = jnp.clip(arg, 0, _A - 1)
            cyc = jnp.clip(cyi, 0, _H - 1)
            cxc = jnp.clip(cxi, 0, _W - 1)
            rbase = ((b * _A + argc) * 5) * _H + cyc
            for k in range(4):
                idx_ref[k, sl] = rbase + k * _H
            xbuf_ref[sl] = cxc

        gathers = [
            pltpu.async_copy(out_hbm.at[idx_ref.at[k]], rowbuf_ref.at[k], sem)
            for k in range(4)
        ]

        cells = [cell_ref[pl.ds(ci * _L, _L)] for ci in range(_NCH)]
        killed = [jnp.zeros((16,), jnp.bool_) for _ in range(_NCH)]
        for cj in range(_NCH):
            cv = cells[cj]
            for q in range(_L):
                bq = jnp.broadcast_to(cv[q], (16,))
                killed[cj] = killed[cj] | ((cv == bq) & (lanes < q))
                for ci in range(cj):
                    killed[ci] = killed[ci] | (cells[ci] == bq)
        for ci in range(_NCH):
            sl = pl.ds(ci * _L, _L)
            win_c = (mask_ref[sl] != 0) & jnp.logical_not(killed[ci])
            mask_ref[sl] = win_c.astype(jnp.int32)

        for g in gathers:
            g.wait()

        acc = jnp.zeros((16,), jnp.float32)
        cnt = jnp.zeros((16,), jnp.int32)
        for c in range(_NCH):
            sl = pl.ds(c * _L, _L)
            gidx = lanes + (c * _L)
            xv = xbuf_ref[sl]
            win = mask_ref[sl] != 0
            sx = sbuf_ref[0, sl]
            sy = sbuf_ref[1, sl]
            sw = sbuf_ref[2, sl]
            sh = sbuf_ref[3, sl]
            k0 = jnp.zeros((16,), jnp.int32)
            p0 = plsc.load_gather(rowbuf_ref, [k0, gidx, xv])
            p1 = plsc.load_gather(rowbuf_ref, [k0 + 1, gidx, xv])
            p2 = plsc.load_gather(rowbuf_ref, [k0 + 2, gidx, xv])
            p3 = plsc.load_gather(rowbuf_ref, [k0 + 3, gidx, xv])
            d0 = p0 - sx
            d1 = p1 - sy
            d2 = _rsqrt(p2) - _rsqrt(sw)
            d3 = _rsqrt(p3) - _rsqrt(sh)
            contrib = d0 * d0 + d1 * d1 + d2 * d2 + d3 * d3
            acc = acc + jnp.where(win, contrib, 0.0)
            cnt = cnt + win.astype(jnp.int32)

        tot_v = jnp.broadcast_to(jnp.sum(acc), (16,))
        n_v = jnp.broadcast_to(jnp.sum(cnt), (16,)).astype(jnp.float32)
        n_v = jnp.maximum(n_v, 1.0)
        loss_ref[...] = tot_v / (2.0 * n_v)
        pltpu.sync_copy(loss_ref, part_shr.at[pl.ds(b * 16, 16)])


def _body(out_hbm, anch_hbm, tgt_hbm, res_hbm,
          tgt_ref, anch_ref, cell_ref, sbuf_ref, mask_ref,
          idx_ref, xbuf_ref, rowbuf_ref, loss_ref, part_shr, tmp_ref, res_ref, sem):
    cid = lax.axis_index("c")
    sid = lax.axis_index("s")
    active = jnp.logical_and(cid == 0, sid < _B)

    _main_image(active, sid, out_hbm, anch_hbm, tgt_hbm,
                tgt_ref, anch_ref, cell_ref, sbuf_ref, mask_ref,
                idx_ref, xbuf_ref, rowbuf_ref, loss_ref, part_shr, sem)

    plsc.subcore_barrier()

    @pl.when(jnp.logical_and(cid == 0, sid == 0))
    def _reduce():
        facc = jnp.zeros((16,), jnp.float32)
        for bb in range(_B):
            pltpu.sync_copy(part_shr.at[pl.ds(bb * 16, 16)], tmp_ref)
            facc = facc + tmp_ref[...]
        res_ref[...] = facc * (1.0 / _B)
        pltpu.sync_copy(res_ref, res_hbm)


def kernel(output, anchors, targets):
    out_t = jnp.transpose(output, (0, 1, 4, 2, 3))
    out2 = out_t.reshape(_B * _A * 5 * _H, _W)
    tgt_t = jnp.transpose(targets, (2, 0, 1))
    anch_t = jnp.pad(jnp.transpose(anchors, (1, 0)), ((0, 0), (0, 16 - _A)))
    mesh = plsc.VectorSubcoreMesh(core_axis_name="c", subcore_axis_name="s")
    fn = functools.partial(
        pl.kernel,
        out_type=jax.ShapeDtypeStruct((16,), jnp.float32),
        mesh=mesh,
        compiler_params=pltpu.CompilerParams(needs_layout_passes=False),
        scratch_types=[
            pltpu.VMEM((5, _T), jnp.float32),
            pltpu.VMEM((2, 16), jnp.float32),
            pltpu.VMEM((_T,), jnp.int32),
            pltpu.VMEM((4, _T), jnp.float32),
            pltpu.VMEM((_T,), jnp.int32),
            pltpu.VMEM((4, _T), jnp.int32),
            pltpu.VMEM((_T,), jnp.int32),
            pltpu.VMEM((4, _T, _W), jnp.float32),
            pltpu.VMEM((16,), jnp.float32),
            pltpu.VMEM_SHARED((_B * 16,), jnp.float32),
            pltpu.VMEM((16,), jnp.float32),
            pltpu.VMEM((16,), jnp.float32),
            pltpu.SemaphoreType.DMA,
        ],
    )(_body)
    res = fn(out2, anch_t, tgt_t)
    return res[0]


# speedup vs baseline: 76.1471x; 1.0903x over previous
import functools

import jax
import jax.numpy as jnp
from jax import lax
from jax.experimental import pallas as pl
from jax.experimental.pallas import tpu as pltpu
from jax.experimental.pallas import tpu_sc as plsc

_B, _A, _H, _W = 8, 9, 128, 128
_T = 128
_L = 16
_NCH = _T // _L
_DUMP = _A * _H * _W


def _rsqrt(x):
    i = plsc.bitcast(x, jnp.int32)
    i = 0x5F3759DF - jnp.right_shift(i, 1)
    y = plsc.bitcast(i, jnp.float32)
    for _ in range(3):
        y = y * (1.5 - 0.5 * x * y * y)
    return y


def _main_image(active, b, out_hbm, anch_hbm, tgt_hbm,
                tgt_ref, anch_ref, cell_ref, sbuf_ref, mask_ref,
                idx_ref, pred_ref, loss_ref, part_shr, sem):
    @pl.when(active)
    def _main():
        pltpu.sync_copy(tgt_hbm.at[pl.ds(0, 5), b], tgt_ref)
        pltpu.sync_copy(anch_hbm, anch_ref)

        lanes = lax.iota(jnp.int32, 16)
        aw_all = anch_ref[0, pl.ds(0, 16)]
        ah_all = anch_ref[1, pl.ds(0, 16)]
        ax1s, ay1s, areas = [], [], []
        for a in range(_A):
            awv = jnp.broadcast_to(aw_all[a], (16,))
            ahv = jnp.broadcast_to(ah_all[a], (16,))
            ax1 = awv * 0.5
            ay1 = ahv * 0.5
            ax0 = 0.0 - ax1
            ay0 = 0.0 - ay1
            ax1s.append(ax1)
            ay1s.append(ay1)
            areas.append((ax1 - ax0) * (ay1 - ay0))
        for c in range(_NCH):
            sl = pl.ds(c * _L, _L)
            t1 = tgt_ref[1, sl]
            t2 = tgt_ref[2, sl]
            t3 = tgt_ref[3, sl]
            t4 = tgt_ref[4, sl]
            valid = jnp.logical_not(
                (t1 == 0.0) & (t2 == 0.0) & (t3 == 0.0) & (t4 == 0.0))
            sx = t1 * float(_W)
            sy = t2 * float(_H)
            sw = t3 * float(_W)
            sh = t4 * float(_H)
            cxi = sx.astype(jnp.int32)
            cyi = sy.astype(jnp.int32)
            cxf = cxi.astype(jnp.float32)
            cyf = cyi.astype(jnp.float32)
            ox = sx - (cxf + 0.5)
            oy = sy - (cyf + 0.5)
            hw = sw * 0.5
            hh = sh * 0.5
            tx0 = ox - hw
            ty0 = oy - hh
            tx1 = ox + hw
            ty1 = oy + hh
            area_t = (tx1 - tx0) * (ty1 - ty0)
            best = jnp.full((16,), -jnp.inf, jnp.float32)
            arg = jnp.zeros((16,), jnp.int32)
            for a in range(_A):
                ax1 = ax1s[a]
                ay1 = ay1s[a]
                ax0 = 0.0 - ax1
                ay0 = 0.0 - ay1
                area_a = areas[a]
                x0 = jnp.maximum(tx0, ax0)
                y0 = jnp.maximum(ty0, ay0)
                x1 = jnp.minimum(tx1, ax1)
                y1 = jnp.minimum(ty1, ay1)
                flag = ((x0 < x1) & (y0 < y1)).astype(jnp.float32)
                inter = (x1 - x0) * (y1 - y0) * flag
                comb = area_t + area_a
                iou = inter / (comb - inter)
                upd = iou > best
                arg = jnp.where(upd, a, arg)
                best = jnp.where(upd, iou, best)
            maskv = (best > -1.0) & valid
            cell = (arg * _H + cyi) * _W + cxi
            cell = jnp.where(maskv, cell, _DUMP)
            cell_ref[sl] = cell
            mask_ref[sl] = maskv.astype(jnp.int32)
            sbuf_ref[0, sl] = sx
            sbuf_ref[1, sl] = sy
            sbuf_ref[2, sl] = sw
            sbuf_ref[3, sl] = sh
            argc = jnp.clip(arg, 0, _A - 1)
            cyc = jnp.clip(cyi, 0, _H - 1)
            cxc = jnp.clip(cxi, 0, _W - 1)
            fbase = (((b * _A + argc) * 5) * _H + cyc) * _W + cxc
            for k in range(4):
                idx_ref[k, sl] = fbase + k * (_H * _W)

        gathers = [
            pltpu.async_copy(out_hbm.at[idx_ref.at[k]], pred_ref.at[k], sem)
            for k in range(4)
        ]

        cells = [cell_ref[pl.ds(ci * _L, _L)] for ci in range(_NCH)]
        killed = [jnp.zeros((16,), jnp.bool_) for _ in range(_NCH)]
        for cj in range(_NCH):
            cv = cells[cj]
            for q in range(_L):
                bq = jnp.broadcast_to(cv[q], (16,))
                killed[cj] = killed[cj] | ((cv == bq) & (lanes < q))
                for ci in range(cj):
                    killed[ci] = killed[ci] | (cells[ci] == bq)
        for ci in range(_NCH):
            sl = pl.ds(ci * _L, _L)
            win_c = (mask_ref[sl] != 0) & jnp.logical_not(killed[ci])
            mask_ref[sl] = win_c.astype(jnp.int32)

        for g in gathers:
            g.wait()

        acc = jnp.zeros((16,), jnp.float32)
        cnt = jnp.zeros((16,), jnp.int32)
        for c in range(_NCH):
            sl = pl.ds(c * _L, _L)
            win = mask_ref[sl] != 0
            sx = sbuf_ref[0, sl]
            sy = sbuf_ref[1, sl]
            sw = sbuf_ref[2, sl]
            sh = sbuf_ref[3, sl]
            p0 = pred_ref[0, sl]
            p1 = pred_ref[1, sl]
            p2 = pred_ref[2, sl]
            p3 = pred_ref[3, sl]
            d0 = p0 - sx
            d1 = p1 - sy
            d2 = _rsqrt(p2) - _rsqrt(sw)
            d3 = _rsqrt(p3) - _rsqrt(sh)
            contrib = d0 * d0 + d1 * d1 + d2 * d2 + d3 * d3
            acc = acc + jnp.where(win, contrib, 0.0)
            cnt = cnt + win.astype(jnp.int32)

        tot_v = jnp.broadcast_to(jnp.sum(acc), (16,))
        n_v = jnp.broadcast_to(jnp.sum(cnt), (16,)).astype(jnp.float32)
        n_v = jnp.maximum(n_v, 1.0)
        loss_ref[...] = tot_v / (2.0 * n_v)
        pltpu.sync_copy(loss_ref, part_shr.at[pl.ds(b * 16, 16)])


def _body(out_hbm, anch_hbm, tgt_hbm, res_hbm,
          tgt_ref, anch_ref, cell_ref, sbuf_ref, mask_ref,
          idx_ref, pred_ref, loss_ref, part_shr, tmp_ref, res_ref, sem):
    cid = lax.axis_index("c")
    sid = lax.axis_index("s")
    active = jnp.logical_and(cid == 0, sid < _B)

    _main_image(active, sid, out_hbm, anch_hbm, tgt_hbm,
                tgt_ref, anch_ref, cell_ref, sbuf_ref, mask_ref,
                idx_ref, pred_ref, loss_ref, part_shr, sem)

    plsc.subcore_barrier()

    @pl.when(jnp.logical_and(cid == 0, sid == 0))
    def _reduce():
        facc = jnp.zeros((16,), jnp.float32)
        for bb in range(_B):
            pltpu.sync_copy(part_shr.at[pl.ds(bb * 16, 16)], tmp_ref)
            facc = facc + tmp_ref[...]
        res_ref[...] = facc * (1.0 / _B)
        pltpu.sync_copy(res_ref, res_hbm)


def kernel(output, anchors, targets):
    out_t = jnp.transpose(output, (0, 1, 4, 2, 3))
    out1 = out_t.reshape(_B * _A * 5 * _H * _W)
    tgt_t = jnp.transpose(targets, (2, 0, 1))
    anch_t = jnp.pad(jnp.transpose(anchors, (1, 0)), ((0, 0), (0, 16 - _A)))
    mesh = plsc.VectorSubcoreMesh(core_axis_name="c", subcore_axis_name="s")
    fn = functools.partial(
        pl.kernel,
        out_type=jax.ShapeDtypeStruct((16,), jnp.float32),
        mesh=mesh,
        compiler_params=pltpu.CompilerParams(needs_layout_passes=False),
        scratch_types=[
            pltpu.VMEM((5, _T), jnp.float32),
            pltpu.VMEM((2, 16), jnp.float32),
            pltpu.VMEM((_T,), jnp.int32),
            pltpu.VMEM((4, _T), jnp.float32),
            pltpu.VMEM((_T,), jnp.int32),
            pltpu.VMEM((4, _T), jnp.int32),
            pltpu.VMEM((4, _T), jnp.float32),
            pltpu.VMEM((16,), jnp.float32),
            pltpu.VMEM_SHARED((_B * 16,), jnp.float32),
            pltpu.VMEM((16,), jnp.float32),
            pltpu.VMEM((16,), jnp.float32),
            pltpu.SemaphoreType.DMA,
        ],
    )(_body)
    res = fn(out1, anch_t, tgt_t)
    return res[0]


# speedup vs baseline: 78.0998x; 1.0256x over previous
import functools

import jax
import jax.numpy as jnp
from jax import lax
from jax.experimental import pallas as pl
from jax.experimental.pallas import tpu as pltpu
from jax.experimental.pallas import tpu_sc as plsc

_B, _A, _H, _W = 8, 9, 128, 128
_T = 128
_L = 16
_NCH = _T // _L
_DUMP = _A * _H * _W


def _rsqrt(x):
    i = plsc.bitcast(x, jnp.int32)
    i = 0x5F3759DF - jnp.right_shift(i, 1)
    y = plsc.bitcast(i, jnp.float32)
    for _ in range(3):
        y = y * (1.5 - 0.5 * x * y * y)
    return y


def _main_image(active, b, out_hbm, anch_hbm, tgt_hbm,
                tgt_ref, anch_ref, cell_ref, sbuf_ref, mask_ref,
                idx_ref, pred_ref, loss_ref, part_shr, sem):
    @pl.when(active)
    def _main():
        pltpu.sync_copy(tgt_hbm.at[pl.ds(0, 5), b], tgt_ref)
        pltpu.sync_copy(anch_hbm, anch_ref)

        lanes = lax.iota(jnp.int32, 16)
        aw_all = anch_ref[0, pl.ds(0, 16)]
        ah_all = anch_ref[1, pl.ds(0, 16)]
        ax1s, ay1s, areas = [], [], []
        for a in range(_A):
            awv = jnp.broadcast_to(aw_all[a], (16,))
            ahv = jnp.broadcast_to(ah_all[a], (16,))
            ax1 = awv * 0.5
            ay1 = ahv * 0.5
            ax0 = 0.0 - ax1
            ay0 = 0.0 - ay1
            ax1s.append(ax1)
            ay1s.append(ay1)
            areas.append((ax1 - ax0) * (ay1 - ay0))
        for c in range(_NCH):
            sl = pl.ds(c * _L, _L)
            t1 = tgt_ref[1, sl]
            t2 = tgt_ref[2, sl]
            t3 = tgt_ref[3, sl]
            t4 = tgt_ref[4, sl]
            valid = jnp.logical_not(
                (t1 == 0.0) & (t2 == 0.0) & (t3 == 0.0) & (t4 == 0.0))
            sx = t1 * float(_W)
            sy = t2 * float(_H)
            sw = t3 * float(_W)
            sh = t4 * float(_H)
            cxi = sx.astype(jnp.int32)
            cyi = sy.astype(jnp.int32)
            cxf = cxi.astype(jnp.float32)
            cyf = cyi.astype(jnp.float32)
            ox = sx - (cxf + 0.5)
            oy = sy - (cyf + 0.5)
            hw = sw * 0.5
            hh = sh * 0.5
            tx0 = ox - hw
            ty0 = oy - hh
            tx1 = ox + hw
            ty1 = oy + hh
            area_t = (tx1 - tx0) * (ty1 - ty0)
            best = jnp.full((16,), -jnp.inf, jnp.float32)
            arg = jnp.zeros((16,), jnp.int32)
            for a in range(_A):
                ax1 = ax1s[a]
                ay1 = ay1s[a]
                ax0 = 0.0 - ax1
                ay0 = 0.0 - ay1
                area_a = areas[a]
                x0 = jnp.maximum(tx0, ax0)
                y0 = jnp.maximum(ty0, ay0)
                x1 = jnp.minimum(tx1, ax1)
                y1 = jnp.minimum(ty1, ay1)
                flag = ((x0 < x1) & (y0 < y1)).astype(jnp.float32)
                inter = (x1 - x0) * (y1 - y0) * flag
                comb = area_t + area_a
                iou = inter / (comb - inter)
                upd = iou > best
                arg = jnp.where(upd, a, arg)
                best = jnp.where(upd, iou, best)
            maskv = (best > -1.0) & valid
            cell = (arg * _H + cyi) * _W + cxi
            cell = jnp.where(maskv, cell, _DUMP)
            cell_ref[sl] = cell
            mask_ref[sl] = maskv.astype(jnp.int32)
            sbuf_ref[0, sl] = sx
            sbuf_ref[1, sl] = sy
            sbuf_ref[2, sl] = sw
            sbuf_ref[3, sl] = sh
            argc = jnp.clip(arg, 0, _A - 1)
            cyc = jnp.clip(cyi, 0, _H - 1)
            cxc = jnp.clip(cxi, 0, _W - 1)
            fbase = (((b * _A + argc) * 5) * _H + cyc) * _W + cxc
            for k in range(4):
                idx_ref[k, sl] = fbase + k * (_H * _W)

        gathers = [
            pltpu.async_copy(out_hbm.at[idx_ref.at[k]], pred_ref.at[k], sem)
            for k in range(4)
        ]

        cells = [cell_ref[pl.ds(ci * _L, _L)] for ci in range(_NCH)]
        killed = [jnp.zeros((16,), jnp.bool_) for _ in range(_NCH)]
        for cj in range(_NCH):
            cv = cells[cj]
            for q in range(_L):
                bq = jnp.broadcast_to(cv[q], (16,))
                killed[cj] = killed[cj] | ((cv == bq) & (lanes < q))
                for ci in range(cj):
                    killed[ci] = killed[ci] | (cells[ci] == bq)
        for ci in range(_NCH):
            sl = pl.ds(ci * _L, _L)
            win_c = (mask_ref[sl] != 0) & jnp.logical_not(killed[ci])
            mask_ref[sl] = win_c.astype(jnp.int32)

        for g in gathers:
            g.wait()

        acc = jnp.zeros((16,), jnp.float32)
        cnt = jnp.zeros((16,), jnp.int32)
        for c in range(_NCH):
            sl = pl.ds(c * _L, _L)
            win = mask_ref[sl] != 0
            sx = sbuf_ref[0, sl]
            sy = sbuf_ref[1, sl]
            sw = sbuf_ref[2, sl]
            sh = sbuf_ref[3, sl]
            p0 = pred_ref[0, sl]
            p1 = pred_ref[1, sl]
            p2 = pred_ref[2, sl]
            p3 = pred_ref[3, sl]
            d0 = p0 - sx
            d1 = p1 - sy
            d2 = _rsqrt(p2) - _rsqrt(sw)
            d3 = _rsqrt(p3) - _rsqrt(sh)
            contrib = d0 * d0 + d1 * d1 + d2 * d2 + d3 * d3
            acc = acc + jnp.where(win, contrib, 0.0)
            cnt = cnt + win.astype(jnp.int32)

        tot_v = jnp.broadcast_to(jnp.sum(acc), (16,))
        n_v = jnp.broadcast_to(jnp.sum(cnt), (16,)).astype(jnp.float32)
        n_v = jnp.maximum(n_v, 1.0)
        loss_ref[...] = tot_v / (2.0 * n_v)
        pltpu.sync_copy(loss_ref, part_shr.at[pl.ds(b * 16, 16)])


def _body(out_hbm, anch_hbm, tgt_hbm, res_hbm,
          tgt_ref, anch_ref, cell_ref, sbuf_ref, mask_ref,
          idx_ref, pred_ref, loss_ref, part_shr, tmp_ref, res_ref, sem):
    cid = lax.axis_index("c")
    sid = lax.axis_index("s")
    active = jnp.logical_and(cid == 0, sid < _B)

    _main_image(active, sid, out_hbm, anch_hbm, tgt_hbm,
                tgt_ref, anch_ref, cell_ref, sbuf_ref, mask_ref,
                idx_ref, pred_ref, loss_ref, part_shr, sem)

    plsc.subcore_barrier()

    @pl.when(jnp.logical_and(cid == 0, sid == 0))
    def _reduce():
        pltpu.sync_copy(part_shr, tmp_ref)
        facc = jnp.zeros((16,), jnp.float32)
        for bb in range(_B):
            facc = facc + tmp_ref[pl.ds(bb * 16, 16)]
        res_ref[...] = facc * (1.0 / _B)
        pltpu.sync_copy(res_ref, res_hbm)


def kernel(output, anchors, targets):
    out_t = jnp.transpose(output, (0, 1, 4, 2, 3))
    out1 = out_t.reshape(_B * _A * 5 * _H * _W)
    tgt_t = jnp.transpose(targets, (2, 0, 1))
    anch_t = jnp.pad(jnp.transpose(anchors, (1, 0)), ((0, 0), (0, 16 - _A)))
    mesh = plsc.VectorSubcoreMesh(core_axis_name="c", subcore_axis_name="s")
    fn = functools.partial(
        pl.kernel,
        out_type=jax.ShapeDtypeStruct((16,), jnp.float32),
        mesh=mesh,
        compiler_params=pltpu.CompilerParams(needs_layout_passes=False),
        scratch_types=[
            pltpu.VMEM((5, _T), jnp.float32),
            pltpu.VMEM((2, 16), jnp.float32),
            pltpu.VMEM((_T,), jnp.int32),
            pltpu.VMEM((4, _T), jnp.float32),
            pltpu.VMEM((_T,), jnp.int32),
            pltpu.VMEM((4, _T), jnp.int32),
            pltpu.VMEM((4, _T), jnp.float32),
            pltpu.VMEM((16,), jnp.float32),
            pltpu.VMEM_SHARED((_B * 16,), jnp.float32),
            pltpu.VMEM((_B * 16,), jnp.float32),
            pltpu.VMEM((16,), jnp.float32),
            pltpu.SemaphoreType.DMA,
        ],
    )(_body)
    res = fn(out1, anch_t, tgt_t)
    return res[0]
